# Initial kernel scaffold; baseline (speedup 1.0000x reference)
import jax
import jax.numpy as jnp
from jax.experimental import pallas as pl


def kernel(value, u, edge_index, W1, b1, W2, b2, W3, b3, W4, b4, W5, b5, W6, b6, W7, b7, W8, b8, W9, b9, W10, b10):
    raise NotImplementedError("write your pallas kernel here")


# speedup vs baseline: 2.7877x; 2.7877x over previous
import functools

import jax
import jax.numpy as jnp
from jax import lax
from jax.experimental import pallas as pl
from jax.experimental.pallas import tpu as pltpu
from jax.experimental.pallas import tpu_sc as plsc

N = 10000
E = 320000

NC = 2
NS = 16
BATCH = 128
E_PAD = 32 * 80 * BATCH
ROWS_PER_TILE = (E_PAD // BATCH) // (NC * NS)
N_ACC = 10112
NZ = N_ACC // NS


def _make_segsum(F):
  mesh = plsc.VectorSubcoreMesh(core_axis_name="c", subcore_axis_name="s")

  @functools.partial(
      pl.kernel,
      out_type=jax.ShapeDtypeStruct((2 * N_ACC, F), jnp.float32),
      mesh=mesh,
      scratch_types=[
          pltpu.VMEM((ROWS_PER_TILE, BATCH), jnp.int32),
          pltpu.VMEM((ROWS_PER_TILE, BATCH), jnp.int32),
          pltpu.VMEM((BATCH, F), jnp.float32),
          pltpu.VMEM_SHARED((N_ACC, F), jnp.float32),
          pltpu.SemaphoreType.DMA,
      ],
      compiler_params=pltpu.CompilerParams(use_tc_tiling_on_sc=False),
  )
  def segsum(x_hbm, src_hbm, dst_hbm, z_hbm, out_hbm, sidx, didx, rows, acc,
             sem):
    c = lax.axis_index("c")
    s = lax.axis_index("s")
    wid = c * NS + s
    pltpu.sync_copy(z_hbm, acc.at[pl.ds(s * NZ, NZ)])
    pltpu.sync_copy(src_hbm.at[pl.ds(wid * ROWS_PER_TILE, ROWS_PER_TILE)], sidx)
    pltpu.sync_copy(dst_hbm.at[pl.ds(wid * ROWS_PER_TILE, ROWS_PER_TILE)], didx)
    plsc.subcore_barrier()

    def step(b, carry):
      pltpu.async_copy(x_hbm.at[sidx.at[b]], rows, sem).wait()
      pltpu.sync_copy(rows, acc.at[didx.at[b]], add=True)
      return carry

    lax.fori_loop(0, ROWS_PER_TILE, step, 0)
    plsc.subcore_barrier()
    pltpu.sync_copy(acc.at[pl.ds(s * NZ, NZ)],
                    out_hbm.at[pl.ds(c * N_ACC + s * NZ, NZ)])

  return segsum


_segsum = {F: _make_segsum(F) for F in (16, 64, 128)}


def _segment_sum(x, src2d, dst2d):
  F = x.shape[1]
  zeros = jnp.zeros((NZ, F), jnp.float32)
  out = _segsum[F](x, src2d, dst2d, zeros)
  return jnp.stack([out[:N], out[N_ACC:N_ACC + N]])


def _tc(body, out_shape, *args):
  return pl.pallas_call(body, out_shape=out_shape)(*args)


def _proj1_body(v_ref, u_ref, w_ref, o_ref):
  h0 = jnp.concatenate([v_ref[...], u_ref[...]], axis=1)
  o_ref[...] = jnp.dot(h0, w_ref[...], preferred_element_type=jnp.float32)


def _layer1_body(s_ref, d_ref, b_ref, h_ref, scale_ref):
  dsum = (d_ref[0] + d_ref[1])[:, 0:1]
  scale = 1.0 / jnp.clip(dsum, 1.0, None)
  h_ref[...] = (s_ref[0] + s_ref[1]) * scale + b_ref[...]
  scale_ref[...] = scale


def _make_mid_body(act, with_w2):
  def body(s_ref, scale_ref, w_ref, b_ref, *rest):
    if with_w2:
      w2_ref, o_ref = rest
    else:
      (o_ref,) = rest
    t = (s_ref[0] + s_ref[1]) * scale_ref[...]
    h = jnp.dot(t, w_ref[...], preferred_element_type=jnp.float32) + b_ref[...]
    if act == "lrelu":
      h = jnp.where(h > 0, h, 0.01 * h)
    elif act == "sigmoid":
      h = jax.nn.sigmoid(h)
    if with_w2:
      h = jnp.dot(h, w2_ref[...], preferred_element_type=jnp.float32)
      h = jnp.broadcast_to(h, (h.shape[0], 16))
    o_ref[...] = h

  return body


def _final_body(s_ref, scale_ref, b_ref, o_ref):
  o_ref[...] = (s_ref[0] + s_ref[1])[:, 0:1] * scale_ref[...] + b_ref[...]


def kernel(value, u, edge_index, W1, b1, W2, b2, W3, b3, W4, b4, W5, b5, W6,
           b6, W7, b7, W8, b8, W9, b9, W10, b10):
  src = edge_index[0]
  dst = edge_index[1]
  pad = E_PAD - E
  src2d = jnp.concatenate([src, jnp.zeros((pad,), jnp.int32)]).reshape(
      E_PAD // BATCH, BATCH)
  dst2d = jnp.concatenate([dst, jnp.full((pad,), N, jnp.int32)]).reshape(
      E_PAD // BATCH, BATCH)

  f32 = jnp.float32
  sds = jax.ShapeDtypeStruct

  d = _segment_sum(jnp.ones((N, 16), f32), src2d, dst2d)

  p1 = _tc(_proj1_body, sds((N, 64), f32), value, u, W1)
  s1 = _segment_sum(p1, src2d, dst2d)
  h1, scale = _tc(_layer1_body, (sds((N, 64), f32), sds((N, 1), f32)), s1, d,
                  b1.reshape(1, 64))

  x = h1
  specs = [
      (W2, b2, "lrelu", None),
      (W3, b3, "lrelu", None),
      (W4, b4, "lrelu", None),
      (W5, b5, None, None),
      (W6, b6, None, None),
      (W7, b7, None, None),
      (W8, b8, None, None),
      (W9, b9, "sigmoid", W10),
  ]
  for W, b, act, W_next in specs:
    fin = x.shape[1]
    s = _segment_sum(x, src2d, dst2d)
    fout = W.shape[1] if W_next is None else 16
    body = _make_mid_body(act, W_next is not None)
    args = (s, scale, W, b.reshape(1, -1))
    if W_next is not None:
      args = args + (W_next,)
    x = _tc(body, sds((N, fout), f32), *args)

  s10 = _segment_sum(x, src2d, dst2d)
  out = _tc(_final_body, sds((N, 1), f32), s10, scale, b10.reshape(1, 1))
  return out


# speedup vs baseline: 3.1526x; 1.1309x over previous
import functools

import jax
import jax.numpy as jnp
from jax import lax
from jax.experimental import pallas as pl
from jax.experimental.pallas import tpu as pltpu
from jax.experimental.pallas import tpu_sc as plsc

N = 10000
E = 320000

NC = 2
NS = 16
BATCH = 128
E_PAD = 32 * 80 * BATCH
ROWS_PER_TILE = (E_PAD // BATCH) // (NC * NS)
N_ACC = 10016
NZ = N_ACC // NS
IC = 16


def _make_segsum(F):
  mesh = plsc.VectorSubcoreMesh(core_axis_name="c", subcore_axis_name="s")


  @functools.partial(
      pl.kernel,
      out_type=jax.ShapeDtypeStruct((2 * N_ACC, F), jnp.float32),
      mesh=mesh,
      scratch_types=[
          pltpu.VMEM((2 * IC, BATCH), jnp.int32),
          pltpu.VMEM((2 * IC, BATCH), jnp.int32),
          pltpu.VMEM((2 * BATCH, F), jnp.float32),
          pltpu.VMEM_SHARED((N_ACC, F), jnp.float32),
          pltpu.SemaphoreType.DMA((2,)),
          pltpu.SemaphoreType.DMA((2,)),
      ],
      compiler_params=pltpu.CompilerParams(use_tc_tiling_on_sc=False),
  )
  def segsum(x_hbm, src_hbm, dst_hbm, z_hbm, out_hbm, sidx, didx, rows, acc,
             sem_g, sem_s):
    c = lax.axis_index("c")
    s = lax.axis_index("s")
    wid = c * NS + s
    base = wid * ROWS_PER_TILE
    pltpu.sync_copy(z_hbm, acc.at[pl.ds(s * NZ, NZ)])

    def load_chunk(chunk):
      half = lax.rem(chunk, 2) * IC
      pltpu.sync_copy(src_hbm.at[pl.ds(base + chunk * IC, IC)],
                      sidx.at[pl.ds(half, IC)])
      pltpu.sync_copy(dst_hbm.at[pl.ds(base + chunk * IC, IC)],
                      didx.at[pl.ds(half, IC)])

    load_chunk(0)
    plsc.subcore_barrier()

    def idx_row(ref, b):
      return ref.at[lax.rem(b // IC, 2) * IC + lax.rem(b, IC)]

    def slot_rows(slot):
      return rows.at[pl.ds(slot * BATCH, BATCH)]

    def fire_gather(b, slot):
      pltpu.async_copy(x_hbm.at[idx_row(sidx, b)], slot_rows(slot),
                       sem_g.at[slot])

    def drain(slot, sem):
      pltpu.make_async_copy(x_hbm.at[pl.ds(0, BATCH)], slot_rows(slot),
                            sem.at[slot]).wait()

    fire_gather(0, 0)

    def step(b, carry):
      slot = lax.rem(b, 2)
      nslot = 1 - slot
      @pl.when(b >= 1)
      def _():
        drain(nslot, sem_s)

      @pl.when((lax.rem(b + 1, IC) == 0) & (b + 1 < ROWS_PER_TILE))
      def _():
        load_chunk((b + 1) // IC)

      @pl.when(b + 1 < ROWS_PER_TILE)
      def _():
        fire_gather(b + 1, nslot)

      drain(slot, sem_g)
      pltpu.async_copy(slot_rows(slot), acc.at[idx_row(didx, b)],
                       sem_s.at[slot], add=True)
      return carry

    lax.fori_loop(0, ROWS_PER_TILE, step, 0)
    drain((ROWS_PER_TILE - 1) % 2, sem_s)
    plsc.subcore_barrier()
    pltpu.sync_copy(acc.at[pl.ds(s * NZ, NZ)],
                    out_hbm.at[pl.ds(c * N_ACC + s * NZ, NZ)])

  return segsum


_segsum = {F: _make_segsum(F) for F in (16, 64, 128)}


def _segment_sum(x, src2d, dst2d):
  F = x.shape[1]
  zeros = jnp.zeros((NZ, F), jnp.float32)
  out = _segsum[F](x, src2d, dst2d, zeros)
  return jnp.stack([out[:N], out[N_ACC:N_ACC + N]])


def _tc(body, out_shape, *args):
  return pl.pallas_call(body, out_shape=out_shape)(*args)


def _proj1_body(v_ref, u_ref, w_ref, o_ref):
  h0 = jnp.concatenate([v_ref[...], u_ref[...]], axis=1)
  o_ref[...] = jnp.dot(h0, w_ref[...], preferred_element_type=jnp.float32)


def _layer1_body(s_ref, d_ref, b_ref, h_ref, scale_ref):
  dsum = (d_ref[0] + d_ref[1])[:, 0:1]
  scale = 1.0 / jnp.clip(dsum, 1.0, None)
  h_ref[...] = (s_ref[0] + s_ref[1]) * scale + b_ref[...]
  scale_ref[...] = scale


def _make_mid_body(act, with_w2):
  def body(s_ref, scale_ref, w_ref, b_ref, *rest):
    if with_w2:
      w2_ref, o_ref = rest
    else:
      (o_ref,) = rest
    t = (s_ref[0] + s_ref[1]) * scale_ref[...]
    h = jnp.dot(t, w_ref[...], preferred_element_type=jnp.float32) + b_ref[...]
    if act == "lrelu":
      h = jnp.where(h > 0, h, 0.01 * h)
    elif act == "sigmoid":
      h = jax.nn.sigmoid(h)
    if with_w2:
      h = jnp.dot(h, w2_ref[...], preferred_element_type=jnp.float32)
      h = jnp.broadcast_to(h, (h.shape[0], 16))
    o_ref[...] = h

  return body


def _final_body(s_ref, scale_ref, b_ref, o_ref):
  o_ref[...] = (s_ref[0] + s_ref[1])[:, 0:1] * scale_ref[...] + b_ref[...]


def kernel(value, u, edge_index, W1, b1, W2, b2, W3, b3, W4, b4, W5, b5, W6,
           b6, W7, b7, W8, b8, W9, b9, W10, b10):
  src = edge_index[0]
  dst = edge_index[1]
  pad = E_PAD - E
  src2d = jnp.concatenate([src, jnp.zeros((pad,), jnp.int32)]).reshape(
      E_PAD // BATCH, BATCH)
  dst2d = jnp.concatenate([dst, jnp.full((pad,), N, jnp.int32)]).reshape(
      E_PAD // BATCH, BATCH)

  f32 = jnp.float32
  sds = jax.ShapeDtypeStruct

  d = _segment_sum(jnp.ones((N, 16), f32), src2d, dst2d)

  p1 = _tc(_proj1_body, sds((N, 64), f32), value, u, W1)
  s1 = _segment_sum(p1, src2d, dst2d)
  h1, scale = _tc(_layer1_body, (sds((N, 64), f32), sds((N, 1), f32)), s1, d,
                  b1.reshape(1, 64))

  x = h1
  specs = [
      (W2, b2, "lrelu", None),
      (W3, b3, "lrelu", None),
      (W4, b4, "lrelu", None),
      (W5, b5, None, None),
      (W6, b6, None, None),
      (W7, b7, None, None),
      (W8, b8, None, None),
      (W9, b9, "sigmoid", W10),
  ]
  for W, b, act, W_next in specs:
    fin = x.shape[1]
    s = _segment_sum(x, src2d, dst2d)
    fout = W.shape[1] if W_next is None else 16
    body = _make_mid_body(act, W_next is not None)
    args = (s, scale, W, b.reshape(1, -1))
    if W_next is not None:
      args = args + (W_next,)
    x = _tc(body, sds((N, fout), f32), *args)

  s10 = _segment_sum(x, src2d, dst2d)
  out = _tc(_final_body, sds((N, 1), f32), s10, scale, b10.reshape(1, 1))
  return out


# speedup vs baseline: 7.7099x; 2.4456x over previous
import functools

import jax
import jax.numpy as jnp
from jax import lax
from jax.experimental import pallas as pl
from jax.experimental.pallas import tpu as pltpu
from jax.experimental.pallas import tpu_sc as plsc

N = 10000
E = 320000

NC = 2
NS = 16
BATCH = 128
E_PAD = 32 * 80 * BATCH
NROWS = E_PAD // BATCH
N_ACC = 10016
NZ = N_ACC // NS
NX = N // NS
IC = 16

_mesh = plsc.VectorSubcoreMesh(core_axis_name="c", subcore_axis_name="s")


def _pipeline(x_ref, dummy_hbm, src_hbm, dst_hbm, sidx, didx, rows, acc,
              sem_g, sem_s, base, nb, F):
  def load_chunk(chunk):
    half = lax.rem(chunk, 2) * IC
    pltpu.sync_copy(src_hbm.at[pl.ds(base + chunk * IC, IC)],
                    sidx.at[pl.ds(half, IC)])
    pltpu.sync_copy(dst_hbm.at[pl.ds(base + chunk * IC, IC)],
                    didx.at[pl.ds(half, IC)])

  load_chunk(0)
  plsc.subcore_barrier()

  def idx_row(ref, b):
    return ref.at[lax.rem(b // IC, 2) * IC + lax.rem(b, IC)]

  def slot_rows(slot):
    return rows.at[pl.ds(slot * BATCH, BATCH)]

  def fire_gather(b, slot):
    pltpu.async_copy(x_ref.at[idx_row(sidx, b)], slot_rows(slot),
                     sem_g.at[slot])

  def drain(slot, sem):
    pltpu.make_async_copy(dummy_hbm.at[pl.ds(0, BATCH)], slot_rows(slot),
                          sem.at[slot]).wait()

  fire_gather(0, 0)

  def step(b, carry):
    slot = lax.rem(b, 2)
    nslot = 1 - slot
    @pl.when(b >= 1)
    def _():
      drain(nslot, sem_s)

    @pl.when((lax.rem(b + 1, IC) == 0) & (b + 1 < nb))
    def _():
      load_chunk((b + 1) // IC)

    @pl.when(b + 1 < nb)
    def _():
      fire_gather(b + 1, nslot)

    drain(slot, sem_g)
    pltpu.async_copy(slot_rows(slot), acc.at[idx_row(didx, b)],
                     sem_s.at[slot], add=True)
    return carry

  lax.fori_loop(0, nb, step, 0)
  drain((nb - 1) % 2, sem_s)
  plsc.subcore_barrier()


def _make_segsum_fs(FH):
  NB = NROWS // NS

  @functools.partial(
      pl.kernel,
      out_type=jax.ShapeDtypeStruct((2 * N_ACC, FH), jnp.float32),
      mesh=_mesh,
      scratch_types=[
          pltpu.VMEM((2 * IC, BATCH), jnp.int32),
          pltpu.VMEM((2 * IC, BATCH), jnp.int32),
          pltpu.VMEM((2 * BATCH, FH), jnp.float32),
          pltpu.VMEM_SHARED((N, FH), jnp.float32),
          pltpu.VMEM_SHARED((N_ACC, FH), jnp.float32),
          pltpu.SemaphoreType.DMA((2,)),
          pltpu.SemaphoreType.DMA((2,)),
      ],
      compiler_params=pltpu.CompilerParams(use_tc_tiling_on_sc=False),
  )
  def segsum(xs_hbm, src_hbm, dst_hbm, z_hbm, out_hbm, sidx, didx, rows, x_sh,
             acc, sem_g, sem_s):
    c = lax.axis_index("c")
    s = lax.axis_index("s")
    pltpu.sync_copy(z_hbm, acc.at[pl.ds(s * NZ, NZ)])
    pltpu.sync_copy(xs_hbm.at[pl.ds(c * N + s * NX, NX)],
                    x_sh.at[pl.ds(s * NX, NX)])
    _pipeline(x_sh, xs_hbm, src_hbm, dst_hbm, sidx, didx, rows, acc, sem_g,
              sem_s, s * NB, NB, FH)
    pltpu.sync_copy(acc.at[pl.ds(s * NZ, NZ)],
                    out_hbm.at[pl.ds(c * N_ACC + s * NZ, NZ)])

  return segsum


def _make_segsum_es(F):
  NB = NROWS // (NC * NS)

  @functools.partial(
      pl.kernel,
      out_type=jax.ShapeDtypeStruct((2 * N_ACC, F), jnp.float32),
      mesh=_mesh,
      scratch_types=[
          pltpu.VMEM((2 * IC, BATCH), jnp.int32),
          pltpu.VMEM((2 * IC, BATCH), jnp.int32),
          pltpu.VMEM((2 * BATCH, F), jnp.float32),
          pltpu.VMEM_SHARED((N_ACC, F), jnp.float32),
          pltpu.SemaphoreType.DMA((2,)),
          pltpu.SemaphoreType.DMA((2,)),
      ],
      compiler_params=pltpu.CompilerParams(use_tc_tiling_on_sc=False),
  )
  def segsum(x_hbm, src_hbm, dst_hbm, z_hbm, out_hbm, sidx, didx, rows, acc,
             sem_g, sem_s):
    c = lax.axis_index("c")
    s = lax.axis_index("s")
    wid = c * NS + s
    pltpu.sync_copy(z_hbm, acc.at[pl.ds(s * NZ, NZ)])
    _pipeline(x_hbm, x_hbm, src_hbm, dst_hbm, sidx, didx, rows, acc, sem_g,
              sem_s, wid * NB, NB, F)
    pltpu.sync_copy(acc.at[pl.ds(s * NZ, NZ)],
                    out_hbm.at[pl.ds(c * N_ACC + s * NZ, NZ)])

  return segsum


_segsum_fs = {FH: _make_segsum_fs(FH) for FH in (32, 64)}
_segsum_es16 = _make_segsum_es(16)


def _segment_sum_fs(xs, src2d, dst2d):
  FH = xs.shape[2]
  zeros = jnp.zeros((NZ, FH), jnp.float32)
  out = _segsum_fs[FH](xs.reshape(2 * N, FH), src2d, dst2d, zeros)
  return jnp.stack([out[:N], out[N_ACC:N_ACC + N]])


def _segment_sum_es16(x, src2d, dst2d):
  zeros = jnp.zeros((NZ, 16), jnp.float32)
  out = _segsum_es16(x, src2d, dst2d, zeros)
  return jnp.stack([out[:N], out[N_ACC:N_ACC + N]])


def _tc(body, out_shape, *args):
  return pl.pallas_call(body, out_shape=out_shape)(*args)


def _split(h):
  fh = h.shape[1] // 2
  return jnp.stack([h[:, :fh], h[:, fh:]])


def _proj1_body(v_ref, u_ref, w_ref, o_ref):
  h0 = jnp.concatenate([v_ref[...], u_ref[...]], axis=1)
  o_ref[...] = _split(jnp.dot(h0, w_ref[...],
                              preferred_element_type=jnp.float32))


def _layer1_body(s_ref, d_ref, b_ref, h_ref, scale_ref):
  dsum = (d_ref[0] + d_ref[1])[:, 0:1]
  scale = 1.0 / jnp.clip(dsum, 1.0, None)
  agg = jnp.concatenate([s_ref[0], s_ref[1]], axis=1)
  h_ref[...] = _split(agg * scale + b_ref[...])
  scale_ref[...] = scale


def _make_mid_body(act, with_w2):
  def body(s_ref, scale_ref, w_ref, b_ref, *rest):
    if with_w2:
      w2_ref, o_ref = rest
    else:
      (o_ref,) = rest
    t = jnp.concatenate([s_ref[0], s_ref[1]], axis=1) * scale_ref[...]
    h = jnp.dot(t, w_ref[...], preferred_element_type=jnp.float32) + b_ref[...]
    if act == "lrelu":
      h = jnp.where(h > 0, h, 0.01 * h)
    elif act == "sigmoid":
      h = jax.nn.sigmoid(h)
    if with_w2:
      h = jnp.dot(h, w2_ref[...], preferred_element_type=jnp.float32)
      o_ref[...] = jnp.broadcast_to(h, (h.shape[0], 16))
    else:
      o_ref[...] = _split(h)

  return body


def _final_body(s_ref, scale_ref, b_ref, o_ref):
  o_ref[...] = (s_ref[0] + s_ref[1])[:, 0:1] * scale_ref[...] + b_ref[...]


def kernel(value, u, edge_index, W1, b1, W2, b2, W3, b3, W4, b4, W5, b5, W6,
           b6, W7, b7, W8, b8, W9, b9, W10, b10):
  src = edge_index[0]
  dst = edge_index[1]
  pad = E_PAD - E
  src2d = jnp.concatenate([src, jnp.zeros((pad,), jnp.int32)]).reshape(
      NROWS, BATCH)
  dst2d = jnp.concatenate([dst, jnp.full((pad,), N, jnp.int32)]).reshape(
      NROWS, BATCH)

  f32 = jnp.float32
  sds = jax.ShapeDtypeStruct

  d = _segment_sum_es16(jnp.ones((N, 16), f32), src2d, dst2d)

  p1 = _tc(_proj1_body, sds((2, N, 32), f32), value, u, W1)
  s1 = _segment_sum_fs(p1, src2d, dst2d)
  h1, scale = _tc(_layer1_body, (sds((2, N, 32), f32), sds((N, 1), f32)), s1,
                  d, b1.reshape(1, 64))

  x = h1
  specs = [
      (W2, b2, "lrelu", None),
      (W3, b3, "lrelu", None),
      (W4, b4, "lrelu", None),
      (W5, b5, None, None),
      (W6, b6, None, None),
      (W7, b7, None, None),
      (W8, b8, None, None),
      (W9, b9, "sigmoid", W10),
  ]
  for W, b, act, W_next in specs:
    s = _segment_sum_fs(x, src2d, dst2d)
    body = _make_mid_body(act, W_next is not None)
    args = (s, scale, W, b.reshape(1, -1))
    if W_next is not None:
      out_shape = sds((N, 16), f32)
      args = args + (W_next,)
    else:
      out_shape = sds((2, N, W.shape[1] // 2), f32)
    x = _tc(body, out_shape, *args)

  s10 = _segment_sum_es16(x, src2d, dst2d)
  out = _tc(_final_body, sds((N, 1), f32), s10, scale, b10.reshape(1, 1))
  return out


# speedup vs baseline: 9.0164x; 1.1694x over previous
import functools

import jax
import jax.numpy as jnp
from jax import lax
from jax.experimental import pallas as pl
from jax.experimental.pallas import tpu as pltpu
from jax.experimental.pallas import tpu_sc as plsc

N = 10000
E = 320000

NC = 2
NS = 16
BATCH = 128
E_PAD = 32 * 80 * BATCH
NROWS = E_PAD // BATCH
N_ACC = 10016
NZ = N_ACC // NS
NX = N // NS
IC = 16

_mesh = plsc.VectorSubcoreMesh(core_axis_name="c", subcore_axis_name="s")


RING = 4


def _pipeline(x_ref, dummy_hbm, src_hbm, dst_hbm, sidx, didx, rows, acc,
              sem_g, sem_s, base, nb, F):
  def load_chunk(chunk):
    half = lax.rem(chunk, 2) * IC
    pltpu.sync_copy(src_hbm.at[pl.ds(base + chunk * IC, IC)],
                    sidx.at[pl.ds(half, IC)])
    pltpu.sync_copy(dst_hbm.at[pl.ds(base + chunk * IC, IC)],
                    didx.at[pl.ds(half, IC)])

  load_chunk(0)
  plsc.subcore_barrier()

  def idx_row(ref, b):
    return ref.at[lax.rem(b // IC, 2) * IC + lax.rem(b, IC)]

  def slot_rows(slot):
    return rows.at[pl.ds(slot * BATCH, BATCH)]

  def fire_gather(b, slot):
    pltpu.async_copy(x_ref.at[idx_row(sidx, b)], slot_rows(slot),
                     sem_g.at[slot])

  def drain(slot, sem):
    pltpu.make_async_copy(dummy_hbm.at[pl.ds(0, BATCH)], slot_rows(slot),
                          sem.at[slot]).wait()

  for b0 in range(RING - 1):
    fire_gather(b0, b0)

  def step(b, carry):
    slot = lax.rem(b, RING)
    nslot = lax.rem(b + 1, RING)
    @pl.when(b >= RING - 1)
    def _():
      drain(nslot, sem_s)

    @pl.when((lax.rem(b + 1, IC) == 0) & (b + 1 < nb))
    def _():
      load_chunk((b + 1) // IC)

    @pl.when((b + 1 >= RING - 1) & (b + 1 < nb))
    def _():
      fire_gather(b + 1, nslot)

    drain(slot, sem_g)
    pltpu.async_copy(slot_rows(slot), acc.at[idx_row(didx, b)],
                     sem_s.at[slot], add=True)
    return carry

  lax.fori_loop(0, nb, step, 0)
  for b in range(nb - RING + 1, nb):
    drain(b % RING, sem_s)
  plsc.subcore_barrier()


def _make_segsum_fs(FH):
  NB = NROWS // NS

  @functools.partial(
      pl.kernel,
      out_type=jax.ShapeDtypeStruct((2 * N_ACC, FH), jnp.float32),
      mesh=_mesh,
      scratch_types=[
          pltpu.VMEM((2 * IC, BATCH), jnp.int32),
          pltpu.VMEM((2 * IC, BATCH), jnp.int32),
          pltpu.VMEM((RING * BATCH, FH), jnp.float32),
          pltpu.VMEM_SHARED((N, FH), jnp.float32),
          pltpu.VMEM_SHARED((N_ACC, FH), jnp.float32),
          pltpu.SemaphoreType.DMA((RING,)),
          pltpu.SemaphoreType.DMA((RING,)),
      ],
      compiler_params=pltpu.CompilerParams(use_tc_tiling_on_sc=False),
  )
  def segsum(xs_hbm, src_hbm, dst_hbm, z_hbm, out_hbm, sidx, didx, rows, x_sh,
             acc, sem_g, sem_s):
    c = lax.axis_index("c")
    s = lax.axis_index("s")
    pltpu.sync_copy(z_hbm, acc.at[pl.ds(s * NZ, NZ)])
    pltpu.sync_copy(xs_hbm.at[pl.ds(c * N + s * NX, NX)],
                    x_sh.at[pl.ds(s * NX, NX)])
    _pipeline(x_sh, xs_hbm, src_hbm, dst_hbm, sidx, didx, rows, acc, sem_g,
              sem_s, s * NB, NB, FH)
    pltpu.sync_copy(acc.at[pl.ds(s * NZ, NZ)],
                    out_hbm.at[pl.ds(c * N_ACC + s * NZ, NZ)])

  return segsum


def _make_segsum_es(F):
  NB = NROWS // (NC * NS)

  @functools.partial(
      pl.kernel,
      out_type=jax.ShapeDtypeStruct((2 * N_ACC, F), jnp.float32),
      mesh=_mesh,
      scratch_types=[
          pltpu.VMEM((2 * IC, BATCH), jnp.int32),
          pltpu.VMEM((2 * IC, BATCH), jnp.int32),
          pltpu.VMEM((RING * BATCH, F), jnp.float32),
          pltpu.VMEM_SHARED((N_ACC, F), jnp.float32),
          pltpu.SemaphoreType.DMA((RING,)),
          pltpu.SemaphoreType.DMA((RING,)),
      ],
      compiler_params=pltpu.CompilerParams(use_tc_tiling_on_sc=False),
  )
  def segsum(x_hbm, src_hbm, dst_hbm, z_hbm, out_hbm, sidx, didx, rows, acc,
             sem_g, sem_s):
    c = lax.axis_index("c")
    s = lax.axis_index("s")
    wid = c * NS + s
    pltpu.sync_copy(z_hbm, acc.at[pl.ds(s * NZ, NZ)])
    _pipeline(x_hbm, x_hbm, src_hbm, dst_hbm, sidx, didx, rows, acc, sem_g,
              sem_s, wid * NB, NB, F)
    pltpu.sync_copy(acc.at[pl.ds(s * NZ, NZ)],
                    out_hbm.at[pl.ds(c * N_ACC + s * NZ, NZ)])

  return segsum


_segsum_fs = {FH: _make_segsum_fs(FH) for FH in (32, 64)}
_segsum_es16 = _make_segsum_es(16)


def _segment_sum_fs(xs, src2d, dst2d):
  FH = xs.shape[2]
  zeros = jnp.zeros((NZ, FH), jnp.float32)
  out = _segsum_fs[FH](xs.reshape(2 * N, FH), src2d, dst2d, zeros)
  return jnp.stack([out[:N], out[N_ACC:N_ACC + N]])


def _segment_sum_es16(x, src2d, dst2d):
  zeros = jnp.zeros((NZ, 16), jnp.float32)
  out = _segsum_es16(x, src2d, dst2d, zeros)
  return jnp.stack([out[:N], out[N_ACC:N_ACC + N]])


def _tc(body, out_shape, *args):
  return pl.pallas_call(body, out_shape=out_shape)(*args)


def _split(h):
  fh = h.shape[1] // 2
  return jnp.stack([h[:, :fh], h[:, fh:]])


def _proj1_body(v_ref, u_ref, w_ref, o_ref):
  h0 = jnp.concatenate([v_ref[...], u_ref[...]], axis=1)
  o_ref[...] = _split(jnp.dot(h0, w_ref[...],
                              preferred_element_type=jnp.float32))


def _layer1_body(s_ref, d_ref, b_ref, h_ref, scale_ref):
  dsum = (d_ref[0] + d_ref[1])[:, 0:1]
  scale = 1.0 / jnp.clip(dsum, 1.0, None)
  agg = jnp.concatenate([s_ref[0], s_ref[1]], axis=1)
  h_ref[...] = _split(agg * scale + b_ref[...])
  scale_ref[...] = scale


def _make_mid_body(act, with_w2):
  def body(s_ref, scale_ref, w_ref, b_ref, *rest):
    if with_w2:
      w2_ref, o_ref = rest
    else:
      (o_ref,) = rest
    t = jnp.concatenate([s_ref[0], s_ref[1]], axis=1) * scale_ref[...]
    h = jnp.dot(t, w_ref[...], preferred_element_type=jnp.float32) + b_ref[...]
    if act == "lrelu":
      h = jnp.where(h > 0, h, 0.01 * h)
    elif act == "sigmoid":
      h = jax.nn.sigmoid(h)
    if with_w2:
      h = jnp.dot(h, w2_ref[...], preferred_element_type=jnp.float32)
      o_ref[...] = jnp.broadcast_to(h, (h.shape[0], 16))
    else:
      o_ref[...] = _split(h)

  return body


def _final_body(s_ref, scale_ref, b_ref, o_ref):
  o_ref[...] = (s_ref[0] + s_ref[1])[:, 0:1] * scale_ref[...] + b_ref[...]


def kernel(value, u, edge_index, W1, b1, W2, b2, W3, b3, W4, b4, W5, b5, W6,
           b6, W7, b7, W8, b8, W9, b9, W10, b10):
  src = edge_index[0]
  dst = edge_index[1]
  pad = E_PAD - E
  src2d = jnp.concatenate([src, jnp.zeros((pad,), jnp.int32)]).reshape(
      NROWS, BATCH)
  dst2d = jnp.concatenate([dst, jnp.full((pad,), N, jnp.int32)]).reshape(
      NROWS, BATCH)

  f32 = jnp.float32
  sds = jax.ShapeDtypeStruct

  d = _segment_sum_es16(jnp.ones((N, 16), f32), src2d, dst2d)

  p1 = _tc(_proj1_body, sds((2, N, 32), f32), value, u, W1)
  s1 = _segment_sum_fs(p1, src2d, dst2d)
  h1, scale = _tc(_layer1_body, (sds((2, N, 32), f32), sds((N, 1), f32)), s1,
                  d, b1.reshape(1, 64))

  x = h1
  specs = [
      (W2, b2, "lrelu", None),
      (W3, b3, "lrelu", None),
      (W4, b4, "lrelu", None),
      (W5, b5, None, None),
      (W6, b6, None, None),
      (W7, b7, None, None),
      (W8, b8, None, None),
      (W9, b9, "sigmoid", W10),
  ]
  for W, b, act, W_next in specs:
    s = _segment_sum_fs(x, src2d, dst2d)
    body = _make_mid_body(act, W_next is not None)
    args = (s, scale, W, b.reshape(1, -1))
    if W_next is not None:
      out_shape = sds((N, 16), f32)
      args = args + (W_next,)
    else:
      out_shape = sds((2, N, W.shape[1] // 2), f32)
    x = _tc(body, out_shape, *args)

  s10 = _segment_sum_es16(x, src2d, dst2d)
  out = _tc(_final_body, sds((N, 1), f32), s10, scale, b10.reshape(1, 1))
  return out


# speedup vs baseline: 10.8796x; 1.2067x over previous
import functools

import jax
import jax.numpy as jnp
from jax import lax
from jax.experimental import pallas as pl
from jax.experimental.pallas import tpu as pltpu
from jax.experimental.pallas import tpu_sc as plsc

N = 10000
E = 320000

NC = 2
NS = 16
BATCH = 128
E_PAD = 32 * 80 * BATCH
NROWS = E_PAD // BATCH
N_ACC = 10016
NZ = N_ACC // NS
NX = N // NS
IC = 16

_mesh = plsc.VectorSubcoreMesh(core_axis_name="c", subcore_axis_name="s")


RING = 4


def _pipeline(x_ref, dummy_hbm, src_hbm, dst_hbm, sidx, didx, rows, acc,
              sem_g, sem_s, base, nb, F):
  def load_chunk(chunk):
    half = lax.rem(chunk, 2) * IC
    pltpu.sync_copy(src_hbm.at[pl.ds(base + chunk * IC, IC)],
                    sidx.at[pl.ds(half, IC)])
    pltpu.sync_copy(dst_hbm.at[pl.ds(base + chunk * IC, IC)],
                    didx.at[pl.ds(half, IC)])

  load_chunk(0)
  plsc.subcore_barrier()

  def idx_row(ref, b):
    return ref.at[lax.rem(b // IC, 2) * IC + lax.rem(b, IC)]

  def slot_rows(slot):
    return rows.at[pl.ds(slot * BATCH, BATCH)]

  def fire_gather(b, slot):
    pltpu.async_copy(x_ref.at[idx_row(sidx, b)], slot_rows(slot),
                     sem_g.at[slot])

  def drain(slot, sem):
    pltpu.make_async_copy(dummy_hbm.at[pl.ds(0, BATCH)], slot_rows(slot),
                          sem.at[slot]).wait()

  for b0 in range(RING - 1):
    fire_gather(b0, b0)

  def step(b, carry):
    slot = lax.rem(b, RING)
    nslot = lax.rem(b + 1, RING)
    @pl.when(b >= RING - 1)
    def _():
      drain(nslot, sem_s)

    @pl.when((lax.rem(b + 1, IC) == 0) & (b + 1 < nb))
    def _():
      load_chunk((b + 1) // IC)

    @pl.when((b + 1 >= RING - 1) & (b + 1 < nb))
    def _():
      fire_gather(b + 1, nslot)

    drain(slot, sem_g)
    pltpu.async_copy(slot_rows(slot), acc.at[idx_row(didx, b)],
                     sem_s.at[slot], add=True)
    return carry

  lax.fori_loop(0, nb, step, 0)
  for b in range(nb - RING + 1, nb):
    drain(b % RING, sem_s)
  plsc.subcore_barrier()


def _make_segsum_fs(F):
  FH = F // 2
  NB = NROWS // NS

  @functools.partial(
      pl.kernel,
      out_type=jax.ShapeDtypeStruct((N_ACC, F), jnp.float32),
      mesh=_mesh,
      scratch_types=[
          pltpu.VMEM((2 * IC, BATCH), jnp.int32),
          pltpu.VMEM((2 * IC, BATCH), jnp.int32),
          pltpu.VMEM((RING * BATCH, FH), jnp.float32),
          pltpu.VMEM_SHARED((N, FH), jnp.float32),
          pltpu.VMEM_SHARED((N_ACC, FH), jnp.float32),
          pltpu.SemaphoreType.DMA((RING,)),
          pltpu.SemaphoreType.DMA((RING,)),
      ],
      compiler_params=pltpu.CompilerParams(use_tc_tiling_on_sc=False),
  )
  def segsum(x_hbm, src_hbm, dst_hbm, z_hbm, out_hbm, sidx, didx, rows, x_sh,
             acc, sem_g, sem_s):
    c = lax.axis_index("c")
    s = lax.axis_index("s")
    pltpu.sync_copy(z_hbm, acc.at[pl.ds(s * NZ, NZ)])
    pltpu.sync_copy(x_hbm.at[pl.ds(s * NX, NX), pl.ds(c * FH, FH)],
                    x_sh.at[pl.ds(s * NX, NX)])
    _pipeline(x_sh, x_hbm, src_hbm, dst_hbm, sidx, didx, rows, acc, sem_g,
              sem_s, s * NB, NB, FH)
    pltpu.sync_copy(acc.at[pl.ds(s * NZ, NZ)],
                    out_hbm.at[pl.ds(s * NZ, NZ), pl.ds(c * FH, FH)])

  return segsum


def _make_segsum_es(F):
  NB = NROWS // (NC * NS)

  @functools.partial(
      pl.kernel,
      out_type=jax.ShapeDtypeStruct((N_ACC, 2 * F), jnp.float32),
      mesh=_mesh,
      scratch_types=[
          pltpu.VMEM((2 * IC, BATCH), jnp.int32),
          pltpu.VMEM((2 * IC, BATCH), jnp.int32),
          pltpu.VMEM((RING * BATCH, F), jnp.float32),
          pltpu.VMEM_SHARED((N_ACC, F), jnp.float32),
          pltpu.SemaphoreType.DMA((RING,)),
          pltpu.SemaphoreType.DMA((RING,)),
      ],
      compiler_params=pltpu.CompilerParams(use_tc_tiling_on_sc=False),
  )
  def segsum(x_hbm, src_hbm, dst_hbm, z_hbm, out_hbm, sidx, didx, rows, acc,
             sem_g, sem_s):
    c = lax.axis_index("c")
    s = lax.axis_index("s")
    wid = c * NS + s
    pltpu.sync_copy(z_hbm, acc.at[pl.ds(s * NZ, NZ)])
    _pipeline(x_hbm, x_hbm, src_hbm, dst_hbm, sidx, didx, rows, acc, sem_g,
              sem_s, wid * NB, NB, F)
    pltpu.sync_copy(acc.at[pl.ds(s * NZ, NZ)],
                    out_hbm.at[pl.ds(s * NZ, NZ), pl.ds(c * F, F)])

  return segsum


_segsum_fs = {F: _make_segsum_fs(F) for F in (64, 128)}
_segsum_es16 = _make_segsum_es(16)


def _segment_sum_fs(x, src2d, dst2d):
  F = x.shape[1]
  zeros = jnp.zeros((NZ, F // 2), jnp.float32)
  return _segsum_fs[F](x, src2d, dst2d, zeros)


def _segment_sum_es16(x, src2d, dst2d):
  zeros = jnp.zeros((NZ, 16), jnp.float32)
  return _segsum_es16(x, src2d, dst2d, zeros)


def _tc(body, out_shape, *args):
  return pl.pallas_call(body, out_shape=out_shape)(*args)


def _proj1_body(v_ref, u_ref, w_ref, o_ref):
  h0 = jnp.concatenate([v_ref[...], u_ref[...]], axis=1)
  o_ref[...] = jnp.dot(h0, w_ref[...], preferred_element_type=jnp.float32)


def _layer1_body(s_ref, d_ref, b_ref, h_ref, scale_ref):
  dsum = (d_ref[0:N, 0:16] + d_ref[0:N, 16:32])[:, 0:1]
  scale = 1.0 / jnp.clip(dsum, 1.0, None)
  h_ref[...] = s_ref[0:N] * scale + b_ref[...]
  scale_ref[...] = scale


def _make_mid_body(act, with_w2):
  def body(s_ref, scale_ref, w_ref, b_ref, *rest):
    if with_w2:
      w2_ref, o_ref = rest
    else:
      (o_ref,) = rest
    t = s_ref[0:N] * scale_ref[...]
    h = jnp.dot(t, w_ref[...], preferred_element_type=jnp.float32) + b_ref[...]
    if act == "lrelu":
      h = jnp.where(h > 0, h, 0.01 * h)
    elif act == "sigmoid":
      h = jax.nn.sigmoid(h)
    if with_w2:
      h = jnp.dot(h, w2_ref[...], preferred_element_type=jnp.float32)
      h = jnp.broadcast_to(h, (h.shape[0], 16))
    o_ref[...] = h

  return body


def _final_body(s_ref, scale_ref, b_ref, o_ref):
  ssum = (s_ref[0:N, 0:16] + s_ref[0:N, 16:32])[:, 0:1]
  o_ref[...] = ssum * scale_ref[...] + b_ref[...]


def kernel(value, u, edge_index, W1, b1, W2, b2, W3, b3, W4, b4, W5, b5, W6,
           b6, W7, b7, W8, b8, W9, b9, W10, b10):
  src = edge_index[0]
  dst = edge_index[1]
  pad = E_PAD - E
  src2d = jnp.concatenate([src, jnp.zeros((pad,), jnp.int32)]).reshape(
      NROWS, BATCH)
  dst2d = jnp.concatenate([dst, jnp.full((pad,), N, jnp.int32)]).reshape(
      NROWS, BATCH)

  f32 = jnp.float32
  sds = jax.ShapeDtypeStruct

  d = _segment_sum_es16(jnp.ones((N, 16), f32), src2d, dst2d)

  p1 = _tc(_proj1_body, sds((N, 64), f32), value, u, W1)
  s1 = _segment_sum_fs(p1, src2d, dst2d)
  h1, scale = _tc(_layer1_body, (sds((N, 64), f32), sds((N, 1), f32)), s1,
                  d, b1.reshape(1, 64))

  x = h1
  specs = [
      (W2, b2, "lrelu", None),
      (W3, b3, "lrelu", None),
      (W4, b4, "lrelu", None),
      (W5, b5, None, None),
      (W6, b6, None, None),
      (W7, b7, None, None),
      (W8, b8, None, None),
      (W9, b9, "sigmoid", W10),
  ]
  for W, b, act, W_next in specs:
    s = _segment_sum_fs(x, src2d, dst2d)
    body = _make_mid_body(act, W_next is not None)
    args = (s, scale, W, b.reshape(1, -1))
    if W_next is not None:
      out_shape = sds((N, 16), f32)
      args = args + (W_next,)
    else:
      out_shape = sds((N, W.shape[1]), f32)
    x = _tc(body, out_shape, *args)

  s10 = _segment_sum_es16(x, src2d, dst2d)
  out = _tc(_final_body, sds((N, 1), f32), s10, scale, b10.reshape(1, 1))
  return out


# speedup vs baseline: 11.0594x; 1.0165x over previous
import functools

import jax
import jax.numpy as jnp
from jax import lax
from jax.experimental import pallas as pl
from jax.experimental.pallas import tpu as pltpu
from jax.experimental.pallas import tpu_sc as plsc

N = 10000
E = 320000

NC = 2
NS = 16
BATCH = 128
E_PAD = 32 * 80 * BATCH
NROWS = E_PAD // BATCH
N_ACC = 10016
NZ = N_ACC // NS
NX = N // NS
IC = 32

_mesh = plsc.VectorSubcoreMesh(core_axis_name="c", subcore_axis_name="s")


RING = 4


def _pipeline(x_ref, dummy_hbm, src_hbm, dst_hbm, sidx, didx, rows, acc,
              sem_g, sem_s, base, nb, F):
  def load_chunk(chunk):
    half = lax.rem(chunk, 2) * IC
    pltpu.sync_copy(src_hbm.at[pl.ds(base + chunk * IC, IC)],
                    sidx.at[pl.ds(half, IC)])
    pltpu.sync_copy(dst_hbm.at[pl.ds(base + chunk * IC, IC)],
                    didx.at[pl.ds(half, IC)])

  load_chunk(0)
  plsc.subcore_barrier()

  def idx_row(ref, b):
    return ref.at[lax.rem(b // IC, 2) * IC + lax.rem(b, IC)]

  def slot_rows(slot):
    return rows.at[pl.ds(slot * BATCH, BATCH)]

  def fire_gather(b, slot):
    pltpu.async_copy(x_ref.at[idx_row(sidx, b)], slot_rows(slot),
                     sem_g.at[slot])

  def drain(slot, sem):
    pltpu.make_async_copy(dummy_hbm.at[pl.ds(0, BATCH)], slot_rows(slot),
                          sem.at[slot]).wait()

  for b0 in range(RING - 1):
    fire_gather(b0, b0)

  def step(b, carry):
    slot = lax.rem(b, RING)
    nslot = lax.rem(b + 1, RING)
    @pl.when(b >= RING - 1)
    def _():
      drain(nslot, sem_s)

    @pl.when((lax.rem(b + 1, IC) == 0) & (b + 1 < nb))
    def _():
      load_chunk((b + 1) // IC)

    @pl.when((b + 1 >= RING - 1) & (b + 1 < nb))
    def _():
      fire_gather(b + 1, nslot)

    drain(slot, sem_g)
    pltpu.async_copy(slot_rows(slot), acc.at[idx_row(didx, b)],
                     sem_s.at[slot], add=True)
    return carry

  lax.fori_loop(0, nb, step, 0)
  for b in range(nb - RING + 1, nb):
    drain(b % RING, sem_s)
  plsc.subcore_barrier()


def _make_segsum_fs(F):
  FH = F // 2
  NB = NROWS // NS

  @functools.partial(
      pl.kernel,
      out_type=jax.ShapeDtypeStruct((N_ACC, F), jnp.float32),
      mesh=_mesh,
      scratch_types=[
          pltpu.VMEM((2 * IC, BATCH), jnp.int32),
          pltpu.VMEM((2 * IC, BATCH), jnp.int32),
          pltpu.VMEM((RING * BATCH, FH), jnp.float32),
          pltpu.VMEM_SHARED((N, FH), jnp.float32),
          pltpu.VMEM_SHARED((N_ACC, FH), jnp.float32),
          pltpu.SemaphoreType.DMA((RING,)),
          pltpu.SemaphoreType.DMA((RING,)),
      ],
      compiler_params=pltpu.CompilerParams(use_tc_tiling_on_sc=False),
  )
  def segsum(x_hbm, src_hbm, dst_hbm, z_hbm, out_hbm, sidx, didx, rows, x_sh,
             acc, sem_g, sem_s):
    c = lax.axis_index("c")
    s = lax.axis_index("s")
    pltpu.sync_copy(z_hbm, acc.at[pl.ds(s * NZ, NZ)])
    pltpu.sync_copy(x_hbm.at[pl.ds(s * NX, NX), pl.ds(c * FH, FH)],
                    x_sh.at[pl.ds(s * NX, NX)])
    _pipeline(x_sh, x_hbm, src_hbm, dst_hbm, sidx, didx, rows, acc, sem_g,
              sem_s, s * NB, NB, FH)
    pltpu.sync_copy(acc.at[pl.ds(s * NZ, NZ)],
                    out_hbm.at[pl.ds(s * NZ, NZ), pl.ds(c * FH, FH)])

  return segsum


def _make_segsum_es(F):
  NB = NROWS // (NC * NS)

  @functools.partial(
      pl.kernel,
      out_type=jax.ShapeDtypeStruct((N_ACC, 2 * F), jnp.float32),
      mesh=_mesh,
      scratch_types=[
          pltpu.VMEM((2 * IC, BATCH), jnp.int32),
          pltpu.VMEM((2 * IC, BATCH), jnp.int32),
          pltpu.VMEM((RING * BATCH, F), jnp.float32),
          pltpu.VMEM_SHARED((N_ACC, F), jnp.float32),
          pltpu.SemaphoreType.DMA((RING,)),
          pltpu.SemaphoreType.DMA((RING,)),
      ],
      compiler_params=pltpu.CompilerParams(use_tc_tiling_on_sc=False),
  )
  def segsum(x_hbm, src_hbm, dst_hbm, z_hbm, out_hbm, sidx, didx, rows, acc,
             sem_g, sem_s):
    c = lax.axis_index("c")
    s = lax.axis_index("s")
    wid = c * NS + s
    pltpu.sync_copy(z_hbm, acc.at[pl.ds(s * NZ, NZ)])
    _pipeline(x_hbm, x_hbm, src_hbm, dst_hbm, sidx, didx, rows, acc, sem_g,
              sem_s, wid * NB, NB, F)
    pltpu.sync_copy(acc.at[pl.ds(s * NZ, NZ)],
                    out_hbm.at[pl.ds(s * NZ, NZ), pl.ds(c * F, F)])

  return segsum


_segsum_fs = {F: _make_segsum_fs(F) for F in (64, 128)}
_segsum_es16 = _make_segsum_es(16)


def _segment_sum_fs(x, src2d, dst2d):
  F = x.shape[1]
  zeros = jnp.zeros((NZ, F // 2), jnp.float32)
  return _segsum_fs[F](x, src2d, dst2d, zeros)


def _segment_sum_es16(x, src2d, dst2d):
  zeros = jnp.zeros((NZ, 16), jnp.float32)
  return _segsum_es16(x, src2d, dst2d, zeros)


def _tc(body, out_shape, *args):
  return pl.pallas_call(body, out_shape=out_shape)(*args)


def _proj1_body(v_ref, u_ref, w_ref, o_ref):
  h0 = jnp.concatenate([v_ref[...], u_ref[...]], axis=1)
  o_ref[...] = jnp.dot(h0, w_ref[...], preferred_element_type=jnp.float32)


def _layer1_body(s_ref, d_ref, b_ref, h_ref, scale_ref):
  dsum = (d_ref[0:N, 0:16] + d_ref[0:N, 16:32])[:, 0:1]
  scale = 1.0 / jnp.clip(dsum, 1.0, None)
  h_ref[...] = s_ref[0:N] * scale + b_ref[...]
  scale_ref[...] = scale


def _make_mid_body(act, with_w2):
  def body(s_ref, scale_ref, w_ref, b_ref, *rest):
    if with_w2:
      w2_ref, o_ref = rest
    else:
      (o_ref,) = rest
    t = s_ref[0:N] * scale_ref[...]
    h = jnp.dot(t, w_ref[...], preferred_element_type=jnp.float32) + b_ref[...]
    if act == "lrelu":
      h = jnp.where(h > 0, h, 0.01 * h)
    elif act == "sigmoid":
      h = jax.nn.sigmoid(h)
    if with_w2:
      h = jnp.dot(h, w2_ref[...], preferred_element_type=jnp.float32)
      h = jnp.broadcast_to(h, (h.shape[0], 16))
    o_ref[...] = h

  return body


def _final_body(s_ref, scale_ref, b_ref, o_ref):
  ssum = (s_ref[0:N, 0:16] + s_ref[0:N, 16:32])[:, 0:1]
  o_ref[...] = ssum * scale_ref[...] + b_ref[...]


def kernel(value, u, edge_index, W1, b1, W2, b2, W3, b3, W4, b4, W5, b5, W6,
           b6, W7, b7, W8, b8, W9, b9, W10, b10):
  src = edge_index[0]
  dst = edge_index[1]
  pad = E_PAD - E
  src2d = jnp.concatenate([src, jnp.zeros((pad,), jnp.int32)]).reshape(
      NROWS, BATCH)
  dst2d = jnp.concatenate([dst, jnp.full((pad,), N, jnp.int32)]).reshape(
      NROWS, BATCH)

  f32 = jnp.float32
  sds = jax.ShapeDtypeStruct

  d = _segment_sum_es16(jnp.ones((N, 16), f32), src2d, dst2d)

  p1 = _tc(_proj1_body, sds((N, 64), f32), value, u, W1)
  s1 = _segment_sum_fs(p1, src2d, dst2d)
  h1, scale = _tc(_layer1_body, (sds((N, 64), f32), sds((N, 1), f32)), s1,
                  d, b1.reshape(1, 64))

  x = h1
  specs = [
      (W2, b2, "lrelu", None),
      (W3, b3, "lrelu", None),
      (W4, b4, "lrelu", None),
      (W5, b5, None, None),
      (W6, b6, None, None),
      (W7, b7, None, None),
      (W8, b8, None, None),
      (W9, b9, "sigmoid", W10),
  ]
  for W, b, act, W_next in specs:
    s = _segment_sum_fs(x, src2d, dst2d)
    body = _make_mid_body(act, W_next is not None)
    args = (s, scale, W, b.reshape(1, -1))
    if W_next is not None:
      out_shape = sds((N, 16), f32)
      args = args + (W_next,)
    else:
      out_shape = sds((N, W.shape[1]), f32)
    x = _tc(body, out_shape, *args)

  s10 = _segment_sum_es16(x, src2d, dst2d)
  out = _tc(_final_body, sds((N, 1), f32), s10, scale, b10.reshape(1, 1))
  return out
